# baseline (speedup 1.0000x reference)
import functools

import jax
import jax.numpy as jnp
from jax import lax
from jax.experimental import pallas as pl
from jax.experimental.pallas import tpu as pltpu
from jax.experimental.pallas import tpu_sc as plsc

N_NODES = 10000
N_EDGES = 160000
D = 256
HALF = 128
N_GRAPHS = 64
N_CLASS = 64

NC = 2
NS = 16
CH = 128
CHUNKS_PER_TILE = 80
E_PAD = NS * CHUNKS_PER_TILE * CH
ROWS_PER_TILE = N_NODES // NS
ACC_ROWS = N_NODES + NS

_PREC = lax.Precision.HIGHEST


def _fc_in_body(x_ref, wt_ref, b_ref, out_ref):
    x = x_ref[...].astype(jnp.bfloat16)
    y = lax.dot_general(x, wt_ref[...], (((1,), (0,)), ((), ())),
                        preferred_element_type=jnp.float32)
    y = (y + b_ref[...]).astype(jnp.bfloat16)
    out_ref[0] = y[:, :HALF]
    out_ref[1] = y[:, HALF:]


def _fc_in(x, wt, b2d, relu_halves):
    blk = 1000
    grid = (N_NODES // blk,)
    if relu_halves:
        in_specs = [pl.BlockSpec((2, blk, HALF), lambda i: (0, i, 0))]
        body = _fc_mid_body
    else:
        in_specs = [pl.BlockSpec((blk, D), lambda i: (i, 0))]
        body = _fc_in_body
    in_specs += [
        pl.BlockSpec((D, D), lambda i: (0, 0)),
        pl.BlockSpec((1, D), lambda i: (0, 0)),
    ]
    return pl.pallas_call(
        body,
        grid=grid,
        in_specs=in_specs,
        out_specs=pl.BlockSpec((2, blk, HALF), lambda i: (0, i, 0)),
        out_shape=jax.ShapeDtypeStruct((2, N_NODES, HALF), jnp.bfloat16),
    )(x, wt, b2d)


def _fc_mid_body(p_ref, wt_ref, b_ref, out_ref):
    h = jnp.concatenate([p_ref[0], p_ref[1]], axis=-1)
    h = jnp.maximum(h, 0)
    y = lax.dot_general(h, wt_ref[...], (((1,), (0,)), ((), ())),
                        preferred_element_type=jnp.float32)
    y = (y + b_ref[...]).astype(jnp.bfloat16)
    out_ref[0] = y[:, :HALF]
    out_ref[1] = y[:, HALF:]


def _prop_body(hsrc, epack, out, acc_sh,
               row_a, row_b, sv_a, sv_b, dv_a, dv_b, blk, gsem):
    c = lax.axis_index("c")
    s = lax.axis_index("s")
    tbase = s * CHUNKS_PER_TILE
    coff = c * N_NODES

    r0 = s * ROWS_PER_TILE
    pltpu.sync_copy(hsrc.at[pl.ds(c * N_NODES + r0, ROWS_PER_TILE)],
                    acc_sh.at[pl.ds(r0, ROWS_PER_TILE)])
    @pl.when(s == 0)
    def _():
        pltpu.sync_copy(hsrc.at[pl.ds(c * N_NODES, NS)],
                        acc_sh.at[pl.ds(N_NODES, NS)])
    plsc.subcore_barrier()

    def unpack(k, sv, dv):
        for v in range(CH // 16):
            sl = pl.ds(v * 16, 16)
            p = blk[k, sl]
            sv[sl] = lax.shift_right_logical(p, 14) + coff
            dv[sl] = lax.bitwise_and(p, 16383)

    def g_issue(sv, row, k):
        pltpu.async_copy(hsrc.at[sv], row, gsem.at[k])

    def g_wait(row, k):
        pltpu.make_async_copy(hsrc.at[pl.ds(0, CH)], row, gsem.at[k]).wait()

    def load_blk(kb):
        pltpu.sync_copy(epack.at[pl.ds(tbase + kb * 8, 8)], blk)

    n_blocks = CHUNKS_PER_TILE // 8

    def outer(kb, carry):
        load_blk(kb)
        for u in range(8):
            unpack(u, sv_a, dv_a)
            pltpu.sync_copy(hsrc.at[sv_a], row_a)
            pltpu.sync_copy(row_a, acc_sh.at[dv_a], add=True)
        return carry

    lax.fori_loop(0, n_blocks, outer, 0)
    plsc.subcore_barrier()

    pltpu.sync_copy(acc_sh.at[pl.ds(r0, ROWS_PER_TILE)],
                    out.at[pl.ds(c * N_NODES + r0, ROWS_PER_TILE)])


@functools.partial(jax.jit, static_argnums=())
def _propagate(hsrc, epack):
    fn = pl.kernel(
        _prop_body,
        out_type=jax.ShapeDtypeStruct((2 * N_NODES, HALF), jnp.bfloat16),
        mesh=plsc.VectorSubcoreMesh(core_axis_name="c", subcore_axis_name="s"),
        scratch_types=(
            [pltpu.VMEM_SHARED((ACC_ROWS, HALF), jnp.bfloat16)]
            + [pltpu.VMEM((CH, HALF), jnp.bfloat16)] * 2
            + [pltpu.VMEM((CH,), jnp.int32)] * 4
            + [pltpu.VMEM((8, CH), jnp.int32)]
            + [pltpu.SemaphoreType.DMA((2,))]
        ),
        compiler_params=pltpu.CompilerParams(use_tc_tiling_on_sc=False),
    )
    return fn(hsrc, epack)


def _head_body(prop_ref, idx_ref, w3t_ref, b3_ref, w4t_ref, b4_ref,
               out_ref, pooled_acc):
    t = pl.program_id(0)
    c = t // 5

    @pl.when(t == 0)
    def _():
        pooled_acc[...] = jnp.zeros_like(pooled_acc)

    h = jnp.maximum(prop_ref[...], 0).astype(jnp.float32)
    idxb = idx_ref[0]
    iota = lax.broadcasted_iota(jnp.int32, (N_GRAPHS, 2000), 0)
    onehot = jnp.where(idxb == iota, 1.0, 0.0)
    part = lax.dot_general(onehot, h, (((1,), (0,)), ((), ())),
                           precision=_PREC, preferred_element_type=jnp.float32)
    csl = pl.ds(c * HALF, HALF)
    pooled_acc[:, csl] += part

    @pl.when(t == 9)
    def _():
        pooled = pooled_acc[...]
        z = lax.dot_general(pooled, w3t_ref[...], (((1,), (0,)), ((), ())),
                            precision=_PREC,
                            preferred_element_type=jnp.float32)
        z = jnp.maximum(z + b3_ref[...], 0.0)
        o = lax.dot_general(z, w4t_ref[...], (((1,), (0,)), ((), ())),
                            precision=_PREC,
                            preferred_element_type=jnp.float32)
        o = o + b4_ref[...]
        m = jnp.max(o, axis=1, keepdims=True)
        lse = m + jnp.log(jnp.sum(jnp.exp(o - m), axis=1, keepdims=True))
        out_ref[...] = o - lse


def _head(prop_flat, idx3, w3t, b3_2d, w4t, b4_2d):
    return pl.pallas_call(
        _head_body,
        grid=(10,),
        in_specs=[
            pl.BlockSpec((2000, HALF), lambda t: (t, 0)),
            pl.BlockSpec((1, 1, 2000), lambda t: (t % 5, 0, 0)),
            pl.BlockSpec((D, D), lambda t: (0, 0)),
            pl.BlockSpec((1, D), lambda t: (0, 0)),
            pl.BlockSpec((D, N_CLASS), lambda t: (0, 0)),
            pl.BlockSpec((1, N_CLASS), lambda t: (0, 0)),
        ],
        out_specs=pl.BlockSpec((N_GRAPHS, N_CLASS), lambda t: (0, 0)),
        out_shape=jax.ShapeDtypeStruct((N_GRAPHS, N_CLASS), jnp.float32),
        scratch_shapes=[pltpu.VMEM((N_GRAPHS, D), jnp.float32)],
    )(prop_flat, idx3, w3t, b3_2d, w4t, b4_2d)


def kernel(x_in, edge_index, idx, W1, b1, W2, b2, W3, b3, W4, b4):
    src = edge_index[0].astype(jnp.int32)
    dst = edge_index[1].astype(jnp.int32)
    pad = E_PAD - N_EDGES
    src_p = jnp.concatenate([src, jnp.zeros((pad,), jnp.int32)])
    dst_p = jnp.concatenate([dst, jnp.full((pad,), N_NODES, jnp.int32)])
    epack = (src_p * 16384 + dst_p).reshape(NS * CHUNKS_PER_TILE, CH)
    idx3 = idx.astype(jnp.int32).reshape(5, 1, 2000)

    h1 = _fc_in(x_in, W1.T.astype(jnp.bfloat16), b1.reshape(1, D),
                relu_halves=False)
    p1 = _propagate(h1.reshape(2 * N_NODES, HALF), epack)
    h2 = _fc_in(p1.reshape(2, N_NODES, HALF), W2.T.astype(jnp.bfloat16),
                b2.reshape(1, D), relu_halves=True)
    p2 = _propagate(h2.reshape(2 * N_NODES, HALF), epack)
    return _head(p2, idx3, W3.T, b3.reshape(1, D), W4.T,
                 b4.reshape(1, N_CLASS))


# speedup vs baseline: 1.2196x; 1.2196x over previous
import functools

import jax
import jax.numpy as jnp
from jax import lax
from jax.experimental import pallas as pl
from jax.experimental.pallas import tpu as pltpu
from jax.experimental.pallas import tpu_sc as plsc

N_NODES = 10000
N_EDGES = 160000
D = 256
HALF = 128
N_GRAPHS = 64
N_CLASS = 64

NC = 2
NS = 16
CH = 128
CHUNKS_PER_TILE = 80
E_PAD = NS * CHUNKS_PER_TILE * CH
ROWS_PER_TILE = N_NODES // NS
ACC_ROWS = N_NODES + NS

_PREC = lax.Precision.HIGHEST


def _fc_in_body(x_ref, wt_ref, b_ref, out_ref):
    x = x_ref[...].astype(jnp.bfloat16)
    y = lax.dot_general(x, wt_ref[...], (((1,), (0,)), ((), ())),
                        preferred_element_type=jnp.float32)
    y = (y + b_ref[...]).astype(jnp.bfloat16)
    out_ref[0] = y[:, :HALF]
    out_ref[1] = y[:, HALF:]


def _fc_in(x, wt, b2d, relu_halves):
    blk = 1000
    grid = (N_NODES // blk,)
    if relu_halves:
        in_specs = [pl.BlockSpec((2, blk, HALF), lambda i: (0, i, 0))]
        body = _fc_mid_body
    else:
        in_specs = [pl.BlockSpec((blk, D), lambda i: (i, 0))]
        body = _fc_in_body
    in_specs += [
        pl.BlockSpec((D, D), lambda i: (0, 0)),
        pl.BlockSpec((1, D), lambda i: (0, 0)),
    ]
    return pl.pallas_call(
        body,
        grid=grid,
        in_specs=in_specs,
        out_specs=pl.BlockSpec((2, blk, HALF), lambda i: (0, i, 0)),
        out_shape=jax.ShapeDtypeStruct((2, N_NODES, HALF), jnp.bfloat16),
    )(x, wt, b2d)


def _fc_mid_body(p_ref, wt_ref, b_ref, out_ref):
    h = jnp.concatenate([p_ref[0], p_ref[1]], axis=-1)
    h = jnp.maximum(h, 0)
    y = lax.dot_general(h, wt_ref[...], (((1,), (0,)), ((), ())),
                        preferred_element_type=jnp.float32)
    y = (y + b_ref[...]).astype(jnp.bfloat16)
    out_ref[0] = y[:, :HALF]
    out_ref[1] = y[:, HALF:]


def _prop_body(hsrc, epack, out, acc_sh,
               row_a, row_b, sv_a, sv_b, dv_a, dv_b, blk, gsem):
    c = lax.axis_index("c")
    s = lax.axis_index("s")
    tbase = s * CHUNKS_PER_TILE
    coff = c * N_NODES

    r0 = s * ROWS_PER_TILE
    pltpu.sync_copy(hsrc.at[pl.ds(c * N_NODES + r0, ROWS_PER_TILE)],
                    acc_sh.at[pl.ds(r0, ROWS_PER_TILE)])
    @pl.when(s == 0)
    def _():
        pltpu.sync_copy(hsrc.at[pl.ds(c * N_NODES, NS)],
                        acc_sh.at[pl.ds(N_NODES, NS)])
    plsc.subcore_barrier()

    def unpack(k, sv, dv):
        for v in range(CH // 16):
            sl = pl.ds(v * 16, 16)
            p = blk[k, sl]
            sv[sl] = lax.shift_right_logical(p, 14) + coff
            dv[sl] = lax.bitwise_and(p, 16383)

    def g_issue(sv, row, k):
        pltpu.async_copy(hsrc.at[sv], row, gsem.at[k])

    def g_wait(row, k):
        pltpu.make_async_copy(hsrc.at[pl.ds(0, CH)], row, gsem.at[k]).wait()

    def load_blk(kb):
        pltpu.sync_copy(epack.at[pl.ds(tbase + kb * 8, 8)], blk)

    load_blk(0)
    unpack(0, sv_a, dv_a)
    g_issue(sv_a, row_a, 0)

    n_blocks = CHUNKS_PER_TILE // 8

    def outer(kb, carry):
        for u in range(8):
            lu = (u + 1) % 8
            sv_l, dv_l, row_l = (sv_b, dv_b, row_b) if u % 2 == 0 \
                else (sv_a, dv_a, row_a)
            sv_s, dv_s, row_s = (sv_a, dv_a, row_a) if u % 2 == 0 \
                else (sv_b, dv_b, row_b)

            if u < 7:
                unpack(lu, sv_l, dv_l)
                g_issue(sv_l, row_l, (u + 1) % 2)
            else:
                @pl.when(kb < n_blocks - 1)
                def _():
                    load_blk(kb + 1)
                    unpack(0, sv_l, dv_l)
                    g_issue(sv_l, row_l, (u + 1) % 2)
            g_wait(row_s, u % 2)
            pltpu.sync_copy(row_s, acc_sh.at[dv_s], add=True)
        return carry

    lax.fori_loop(0, n_blocks, outer, 0)
    plsc.subcore_barrier()

    pltpu.sync_copy(acc_sh.at[pl.ds(r0, ROWS_PER_TILE)],
                    out.at[pl.ds(c * N_NODES + r0, ROWS_PER_TILE)])


@functools.partial(jax.jit, static_argnums=())
def _propagate(hsrc, epack):
    fn = pl.kernel(
        _prop_body,
        out_type=jax.ShapeDtypeStruct((2 * N_NODES, HALF), jnp.bfloat16),
        mesh=plsc.VectorSubcoreMesh(core_axis_name="c", subcore_axis_name="s"),
        scratch_types=(
            [pltpu.VMEM_SHARED((ACC_ROWS, HALF), jnp.bfloat16)]
            + [pltpu.VMEM((CH, HALF), jnp.bfloat16)] * 2
            + [pltpu.VMEM((CH,), jnp.int32)] * 4
            + [pltpu.VMEM((8, CH), jnp.int32)]
            + [pltpu.SemaphoreType.DMA((2,))]
        ),
        compiler_params=pltpu.CompilerParams(use_tc_tiling_on_sc=False),
    )
    return fn(hsrc, epack)


def _head_body(prop_ref, idx_ref, w3t_ref, b3_ref, w4t_ref, b4_ref,
               out_ref, pooled_acc):
    t = pl.program_id(0)
    c = t // 5

    @pl.when(t == 0)
    def _():
        pooled_acc[...] = jnp.zeros_like(pooled_acc)

    h = jnp.maximum(prop_ref[...], 0).astype(jnp.float32)
    idxb = idx_ref[0]
    iota = lax.broadcasted_iota(jnp.int32, (N_GRAPHS, 2000), 0)
    onehot = jnp.where(idxb == iota, 1.0, 0.0)
    part = lax.dot_general(onehot, h, (((1,), (0,)), ((), ())),
                           precision=_PREC, preferred_element_type=jnp.float32)
    csl = pl.ds(c * HALF, HALF)
    pooled_acc[:, csl] += part

    @pl.when(t == 9)
    def _():
        pooled = pooled_acc[...]
        z = lax.dot_general(pooled, w3t_ref[...], (((1,), (0,)), ((), ())),
                            precision=_PREC,
                            preferred_element_type=jnp.float32)
        z = jnp.maximum(z + b3_ref[...], 0.0)
        o = lax.dot_general(z, w4t_ref[...], (((1,), (0,)), ((), ())),
                            precision=_PREC,
                            preferred_element_type=jnp.float32)
        o = o + b4_ref[...]
        m = jnp.max(o, axis=1, keepdims=True)
        lse = m + jnp.log(jnp.sum(jnp.exp(o - m), axis=1, keepdims=True))
        out_ref[...] = o - lse


def _head(prop_flat, idx3, w3t, b3_2d, w4t, b4_2d):
    return pl.pallas_call(
        _head_body,
        grid=(10,),
        in_specs=[
            pl.BlockSpec((2000, HALF), lambda t: (t, 0)),
            pl.BlockSpec((1, 1, 2000), lambda t: (t % 5, 0, 0)),
            pl.BlockSpec((D, D), lambda t: (0, 0)),
            pl.BlockSpec((1, D), lambda t: (0, 0)),
            pl.BlockSpec((D, N_CLASS), lambda t: (0, 0)),
            pl.BlockSpec((1, N_CLASS), lambda t: (0, 0)),
        ],
        out_specs=pl.BlockSpec((N_GRAPHS, N_CLASS), lambda t: (0, 0)),
        out_shape=jax.ShapeDtypeStruct((N_GRAPHS, N_CLASS), jnp.float32),
        scratch_shapes=[pltpu.VMEM((N_GRAPHS, D), jnp.float32)],
    )(prop_flat, idx3, w3t, b3_2d, w4t, b4_2d)


def kernel(x_in, edge_index, idx, W1, b1, W2, b2, W3, b3, W4, b4):
    src = edge_index[0].astype(jnp.int32)
    dst = edge_index[1].astype(jnp.int32)
    pad = E_PAD - N_EDGES
    src_p = jnp.concatenate([src, jnp.zeros((pad,), jnp.int32)])
    dst_p = jnp.concatenate([dst, jnp.full((pad,), N_NODES, jnp.int32)])
    epack = (src_p * 16384 + dst_p).reshape(NS * CHUNKS_PER_TILE, CH)
    idx3 = idx.astype(jnp.int32).reshape(5, 1, 2000)

    h1 = _fc_in(x_in, W1.T.astype(jnp.bfloat16), b1.reshape(1, D),
                relu_halves=False)
    p1 = _propagate(h1.reshape(2 * N_NODES, HALF), epack)
    h2 = _fc_in(p1.reshape(2, N_NODES, HALF), W2.T.astype(jnp.bfloat16),
                b2.reshape(1, D), relu_halves=True)
    p2 = _propagate(h2.reshape(2 * N_NODES, HALF), epack)
    return _head(p2, idx3, W3.T, b3.reshape(1, D), W4.T,
                 b4.reshape(1, N_CLASS))
